# Initial kernel scaffold; baseline (speedup 1.0000x reference)
import jax
import jax.numpy as jnp
from jax.experimental import pallas as pl


def kernel(snp, snp_ids, node_to_gene, gene_edge_index, params):
    raise NotImplementedError("write your pallas kernel here")


# speedup vs baseline: 24.9022x; 24.9022x over previous
import functools

import numpy as np
import jax
import jax.numpy as jnp
from jax import lax
from jax.experimental import pallas as pl
from jax.experimental.pallas import tpu as pltpu
from jax.experimental.pallas import tpu_sc as plsc

NS = 50000
NG = 10000
NB = 8
NA = 50000
NE = 160000
NGP = 10240
NAP = 50176
NROWS = float(NB * NG)
EPS = 1e-5
GBLK = 1000
NSTEP = NG // GBLK

_SC_PARAMS = pltpu.CompilerParams(use_tc_tiling_on_sc=False)

_A_np = np.zeros((64, 512), np.float32)
_B_np = np.zeros((64, 512), np.float32)
for _j in range(8):
    for _b in range(8):
        for _f in range(8):
            _A_np[8 * _j + _b, 64 * _j + 8 * _b + _f] = 1.0
            _B_np[8 * _j + _f, 64 * _j + 8 * _b + _f] = 1.0


def _table(s2, f2, amat, bmat):
    hp = jax.lax.Precision.HIGHEST

    def body(s_ref, f_ref, a_ref, b_ref, o_ref):
        o_ref[...] = (
            jnp.dot(s_ref[...], a_ref[...], precision=hp,
                    preferred_element_type=jnp.float32)
            * jnp.dot(f_ref[...], b_ref[...], precision=hp,
                      preferred_element_type=jnp.float32))
    return pl.pallas_call(
        body,
        out_shape=jax.ShapeDtypeStruct((6250, 512), jnp.float32),
    )(s2, f2, amat, bmat)


@functools.lru_cache(maxsize=None)
def _build_sc_assign():
    mesh = plsc.VectorSubcoreMesh(
        core_axis_name="c", subcore_axis_name="s", num_cores=2, num_subcores=16)

    @functools.partial(
        pl.kernel,
        out_type=jax.ShapeDtypeStruct((2, NGP, 64), jnp.float32),
        mesh=mesh,
        compiler_params=_SC_PARAMS,
        scratch_types=[
            pltpu.VMEM((392,), jnp.int32),
            pltpu.VMEM((392,), jnp.int32),
            pltpu.VMEM((392, 64), jnp.float32),
            pltpu.VMEM((320, 64), jnp.float32),
            pltpu.VMEM_SHARED((NGP, 64), jnp.float32),
            pltpu.SemaphoreType.DMA,
        ],
    )
    def sc_assign(table, ids, genes, z, out, idx_v, gid_v, rows_v, obuf, acc, sem):
        cid = lax.axis_index("c")
        sid = lax.axis_index("s")
        wid = sid * 2 + cid
        pltpu.sync_copy(z, acc.at[pl.ds(sid * 640, 640)])
        plsc.subcore_barrier()
        base = wid * 1568
        for j in range(4):
            sl = pl.ds(base + j * 392, 392)
            pltpu.sync_copy(ids.at[sl], idx_v)
            pltpu.sync_copy(genes.at[sl], gid_v)
            pltpu.async_copy(table.at[idx_v], rows_v, sem).wait()
            pltpu.sync_copy(rows_v, acc.at[gid_v], add=True)
        plsc.subcore_barrier()
        for k in range(2):
            rs = pl.ds(sid * 640 + k * 320, 320)
            pltpu.sync_copy(acc.at[rs], obuf)
            pltpu.sync_copy(obuf, out.at[cid, rs])

    return sc_assign


@functools.lru_cache(maxsize=None)
def _build_sc_edge():
    mesh = plsc.VectorSubcoreMesh(
        core_axis_name="c", subcore_axis_name="s", num_cores=2, num_subcores=16)

    @functools.partial(
        pl.kernel,
        out_type=jax.ShapeDtypeStruct((NB, 2, NGP, 64), jnp.float32),
        mesh=mesh,
        compiler_params=_SC_PARAMS,
        scratch_types=[
            pltpu.VMEM((25, 200), jnp.int32),
            pltpu.VMEM((25, 200), jnp.int32),
            pltpu.VMEM((200, 64), jnp.float32),
            pltpu.VMEM((320, 64), jnp.float32),
            pltpu.VMEM_SHARED((NGP, 64), jnp.float32),
            pltpu.SemaphoreType.DMA,
        ],
    )
    def sc_edge(hcat, srcb, dst, z, out, src_v, dst_v, rows_v, obuf, acc, sem):
        cid = lax.axis_index("c")
        sid = lax.axis_index("s")
        wid = sid * 2 + cid
        base = wid * 5000
        for j in range(25):
            pltpu.sync_copy(dst.at[pl.ds(base + j * 200, 200)], dst_v.at[j])

        def pass_b(b, _):
            pltpu.sync_copy(z, acc.at[pl.ds(sid * 640, 640)])
            plsc.subcore_barrier()
            for j in range(25):
                start = pl.multiple_of(b * NE + base + j * 200, 8)
                pltpu.sync_copy(srcb.at[pl.ds(start, 200)], src_v.at[j])
                pltpu.async_copy(hcat.at[src_v.at[j]], rows_v, sem).wait()
                pltpu.sync_copy(rows_v, acc.at[dst_v.at[j]], add=True)
            plsc.subcore_barrier()
            for k in range(2):
                rs = pl.ds(sid * 640 + k * 320, 320)
                pltpu.sync_copy(acc.at[rs], obuf)
                pltpu.sync_copy(obuf, out.at[b, cid, rs])
            return 0

        lax.fori_loop(0, NB, pass_b, 0)

    return sc_edge


def _bn_scale_shift(ssum, ssq, gvec, betavec):
    mean = ssum / NROWS
    var = ssq / NROWS - mean * mean
    scale = gvec * lax.rsqrt(var + EPS)
    shift = betavec - mean * scale
    return scale, shift


def _ge_a(gf2, w1t, b1):
    def body(gf_ref, w1_ref, b1_ref, x0_ref, s_ref, q_ref):
        @pl.when(pl.program_id(0) == 0)
        def _():
            s_ref[...] = jnp.zeros_like(s_ref)
            q_ref[...] = jnp.zeros_like(q_ref)
        w1 = w1_ref[...]
        gfb = gf_ref[0] + gf_ref[1]
        sacc = s_ref[...]
        qacc = q_ref[...]
        for b in range(8):
            x0 = jnp.dot(gfb[:, 8 * b:8 * b + 8], w1,
                         preferred_element_type=jnp.float32) + b1_ref[...]
            x0_ref[b] = x0
            sacc = sacc + jnp.sum(x0, axis=0, keepdims=True)
            qacc = qacc + jnp.sum(x0 * x0, axis=0, keepdims=True)
        s_ref[...] = sacc
        q_ref[...] = qacc
    return pl.pallas_call(
        body,
        grid=(NSTEP,),
        in_specs=[pl.BlockSpec((2, GBLK, 64), lambda g: (0, g, 0)),
                  pl.BlockSpec((8, 64), lambda g: (0, 0)),
                  pl.BlockSpec((1, 64), lambda g: (0, 0))],
        out_specs=[pl.BlockSpec((8, GBLK, 64), lambda g: (0, g, 0)),
                   pl.BlockSpec((1, 64), lambda g: (0, 0)),
                   pl.BlockSpec((1, 64), lambda g: (0, 0))],
        out_shape=[jax.ShapeDtypeStruct((NB, NG, 64), jnp.float32),
                   jax.ShapeDtypeStruct((1, 64), jnp.float32),
                   jax.ShapeDtypeStruct((1, 64), jnp.float32)],
    )(gf2, w1t, b1)


def _ge_b(x0, ssum, ssq, g1, be1, w2t, b2):
    def body(x0_ref, s_ref, q_ref, g1_ref, be1_ref, w2_ref, b2_ref, o_ref):
        scale, shift = _bn_scale_shift(s_ref[...], q_ref[...], g1_ref[...], be1_ref[...])
        for b in range(8):
            r = jnp.maximum(x0_ref[b] * scale + shift, 0.0)
            o_ref[b] = jnp.dot(r, w2_ref[...], preferred_element_type=jnp.float32) + b2_ref[...]
    return pl.pallas_call(
        body,
        grid=(NSTEP,),
        in_specs=[pl.BlockSpec((8, GBLK, 64), lambda g: (0, g, 0)),
                  pl.BlockSpec((1, 64), lambda g: (0, 0)),
                  pl.BlockSpec((1, 64), lambda g: (0, 0)),
                  pl.BlockSpec((1, 64), lambda g: (0, 0)),
                  pl.BlockSpec((1, 64), lambda g: (0, 0)),
                  pl.BlockSpec((64, 64), lambda g: (0, 0)),
                  pl.BlockSpec((1, 64), lambda g: (0, 0))],
        out_specs=pl.BlockSpec((8, GBLK, 64), lambda g: (0, g, 0)),
        out_shape=jax.ShapeDtypeStruct((NB, NG, 64), jnp.float32),
    )(x0, ssum, ssq, g1, be1, w2t, b2)


def _gin_k1(h, agg, w1t, b1):
    def body(h_ref, a_ref, w1_ref, b1_ref, x1_ref, s_ref, q_ref):
        @pl.when(pl.program_id(0) == 0)
        def _():
            s_ref[...] = jnp.zeros_like(s_ref)
            q_ref[...] = jnp.zeros_like(q_ref)
        w1 = w1_ref[...]
        b1v = b1_ref[...]
        sacc = s_ref[...]
        qacc = q_ref[...]
        for b in range(8):
            pre = h_ref[b] + a_ref[b, 0] + a_ref[b, 1]
            x1 = jnp.dot(pre, w1, preferred_element_type=jnp.float32) + b1v
            x1_ref[b] = x1
            sacc = sacc + jnp.sum(x1, axis=0, keepdims=True)
            qacc = qacc + jnp.sum(x1 * x1, axis=0, keepdims=True)
        s_ref[...] = sacc
        q_ref[...] = qacc
    return pl.pallas_call(
        body,
        grid=(NSTEP,),
        in_specs=[pl.BlockSpec((8, GBLK, 64), lambda g: (0, g, 0)),
                  pl.BlockSpec((8, 2, GBLK, 64), lambda g: (0, 0, g, 0)),
                  pl.BlockSpec((64, 128), lambda g: (0, 0)),
                  pl.BlockSpec((1, 128), lambda g: (0, 0))],
        out_specs=[pl.BlockSpec((8, GBLK, 128), lambda g: (0, g, 0)),
                   pl.BlockSpec((1, 128), lambda g: (0, 0)),
                   pl.BlockSpec((1, 128), lambda g: (0, 0))],
        out_shape=[jax.ShapeDtypeStruct((NB, NG, 128), jnp.float32),
                   jax.ShapeDtypeStruct((1, 128), jnp.float32),
                   jax.ShapeDtypeStruct((1, 128), jnp.float32)],
    )(h, agg, w1t, b1)


def _gin_k2(x1, s1, q1, g1, be1, w2t, b2):
    def body(x1_ref, s_ref, q_ref, g1_ref, be1_ref, w2_ref, b2_ref,
             x2_ref, s2_ref, q2_ref):
        @pl.when(jnp.logical_and(pl.program_id(0) == 0, pl.program_id(1) == 0))
        def _():
            s2_ref[...] = jnp.zeros_like(s2_ref)
            q2_ref[...] = jnp.zeros_like(q2_ref)
        scale, shift = _bn_scale_shift(s_ref[...], q_ref[...], g1_ref[...], be1_ref[...])
        r = jnp.maximum(x1_ref[0] * scale + shift, 0.0)
        x2 = jnp.dot(r, w2_ref[...], preferred_element_type=jnp.float32) + b2_ref[...]
        x2_ref[0] = x2
        s2_ref[...] += jnp.sum(x2, axis=0, keepdims=True)
        q2_ref[...] += jnp.sum(x2 * x2, axis=0, keepdims=True)
    return pl.pallas_call(
        body,
        grid=(NB, NSTEP),
        in_specs=[pl.BlockSpec((1, GBLK, 128), lambda b, g: (b, g, 0)),
                  pl.BlockSpec((1, 128), lambda b, g: (0, 0)),
                  pl.BlockSpec((1, 128), lambda b, g: (0, 0)),
                  pl.BlockSpec((1, 128), lambda b, g: (0, 0)),
                  pl.BlockSpec((1, 128), lambda b, g: (0, 0)),
                  pl.BlockSpec((128, 64), lambda b, g: (0, 0)),
                  pl.BlockSpec((1, 64), lambda b, g: (0, 0))],
        out_specs=[pl.BlockSpec((1, GBLK, 64), lambda b, g: (b, g, 0)),
                   pl.BlockSpec((1, 64), lambda b, g: (0, 0)),
                   pl.BlockSpec((1, 64), lambda b, g: (0, 0))],
        out_shape=[jax.ShapeDtypeStruct((NB, NG, 64), jnp.float32),
                   jax.ShapeDtypeStruct((1, 64), jnp.float32),
                   jax.ShapeDtypeStruct((1, 64), jnp.float32)],
    )(x1, s1, q1, g1, be1, w2t, b2)


def _gin_k3(x2, s2, q2, gg, bb):
    def body(x2_ref, s_ref, q_ref, gg_ref, bb_ref, o_ref):
        scale, shift = _bn_scale_shift(s_ref[...], q_ref[...], gg_ref[...], bb_ref[...])
        for b in range(8):
            o_ref[b] = jnp.maximum(x2_ref[b] * scale + shift, 0.0)
    return pl.pallas_call(
        body,
        grid=(NSTEP,),
        in_specs=[pl.BlockSpec((8, GBLK, 64), lambda g: (0, g, 0)),
                  pl.BlockSpec((1, 64), lambda g: (0, 0)),
                  pl.BlockSpec((1, 64), lambda g: (0, 0)),
                  pl.BlockSpec((1, 64), lambda g: (0, 0)),
                  pl.BlockSpec((1, 64), lambda g: (0, 0))],
        out_specs=pl.BlockSpec((8, GBLK, 64), lambda g: (0, g, 0)),
        out_shape=jax.ShapeDtypeStruct((NB, NG, 64), jnp.float32),
    )(x2, s2, q2, gg, bb)


def _att(h, wkt, bk, qv, wvt, bv):
    def body(h_ref, wk_ref, bk_ref, q_ref, wv_ref, bv_ref, gh_ref):
        @pl.when(pl.program_id(0) == 0)
        def _():
            gh_ref[...] = jnp.zeros_like(gh_ref)
        wk = wk_ref[...]
        wv = wv_ref[...]
        q = q_ref[...]
        for b in range(8):
            hb = h_ref[b]
            keys = jnp.dot(hb, wk, preferred_element_type=jnp.float32) + bk_ref[...]
            logit = jnp.dot(keys, q, preferred_element_type=jnp.float32)
            w = 1.0 / (1.0 + jnp.exp(-logit))
            v = jnp.dot(hb, wv, preferred_element_type=jnp.float32) + bv_ref[...]
            gh_ref[b, :] = gh_ref[b, :] + jnp.sum(v * w, axis=0)
    return pl.pallas_call(
        body,
        grid=(NSTEP,),
        in_specs=[pl.BlockSpec((8, GBLK, 64), lambda g: (0, g, 0)),
                  pl.BlockSpec((64, 64), lambda g: (0, 0)),
                  pl.BlockSpec((1, 64), lambda g: (0, 0)),
                  pl.BlockSpec((64, 1), lambda g: (0, 0)),
                  pl.BlockSpec((64, 64), lambda g: (0, 0)),
                  pl.BlockSpec((1, 64), lambda g: (0, 0))],
        out_specs=pl.BlockSpec((8, 64), lambda g: (0, 0)),
        out_shape=jax.ShapeDtypeStruct((8, 64), jnp.float32),
    )(h, wkt, bk, qv, wvt, bv)


def _head(gh, w1t, b1, g1, be1, w2t, b2, g2, be2, w3t, b3):
    def bn8(x, g, b):
        m = jnp.mean(x, axis=0, keepdims=True)
        v = jnp.mean((x - m) * (x - m), axis=0, keepdims=True)
        return (x - m) * lax.rsqrt(v + EPS) * g + b

    def body(gh_ref, w1_ref, b1_ref, g1_ref, be1_ref, w2_ref, b2_ref,
             g2_ref, be2_ref, w3_ref, b3_ref, o_ref):
        x = jnp.dot(gh_ref[...], w1_ref[...], preferred_element_type=jnp.float32) + b1_ref[...]
        x = jnp.maximum(bn8(x, g1_ref[...], be1_ref[...]), 0.0)
        x = jnp.dot(x, w2_ref[...], preferred_element_type=jnp.float32) + b2_ref[...]
        x = jnp.maximum(bn8(x, g2_ref[...], be2_ref[...]), 0.0)
        o_ref[...] = jnp.dot(x, w3_ref[...], preferred_element_type=jnp.float32) + b3_ref[...]
    return pl.pallas_call(
        body,
        out_shape=jax.ShapeDtypeStruct((8, 1), jnp.float32),
    )(gh, w1t, b1, g1, be1, w2t, b2, g2, be2, w3t, b3)


def kernel(snp, snp_ids, node_to_gene, gene_edge_index, params):
    p = params
    f32 = jnp.float32

    snpT2 = snp.T.reshape(6250, 64).astype(f32)
    filtT2 = p['filters'].T.reshape(6250, 64).astype(f32)
    amat = jnp.asarray(_A_np)
    bmat = jnp.asarray(_B_np)
    table = _table(snpT2, filtT2, amat, bmat).reshape(NS, 64)

    pad_n = NAP - NA
    ids_p = jnp.concatenate(
        [snp_ids.astype(jnp.int32), (jnp.arange(pad_n, dtype=jnp.int32) * 97) % NS])
    genes_p = jnp.concatenate(
        [node_to_gene.astype(jnp.int32),
         NG + (jnp.arange(pad_n, dtype=jnp.int32) % (NGP - NG))])
    z64 = jnp.zeros((640, 64), f32)

    gf2 = _build_sc_assign()(table, ids_p, genes_p, z64)

    row = lambda v: v.reshape(1, -1).astype(f32)
    x0, s0, q0 = _ge_a(gf2, p['ge_w1'].T.astype(f32), row(p['ge_b1']))
    h = _ge_b(x0, s0, q0, row(p['ge_bn_g']), row(p['ge_bn_b']),
              p['ge_w2'].T.astype(f32), row(p['ge_b2']))

    src = gene_edge_index[0].astype(jnp.int32)
    dst = gene_edge_index[1].astype(jnp.int32)
    srcb = (src[None, :] + NG * jnp.arange(NB, dtype=jnp.int32)[:, None]).reshape(-1)

    for l in range(2):
        hcat = h.reshape(NB * NG, 64)
        agg = _build_sc_edge()(hcat, srcb, dst, z64)
        x1, s1, q1 = _gin_k1(
            h, agg, p['gin%d_w1' % l].T.astype(f32), row(p['gin%d_b1' % l]))
        x2, s2, q2 = _gin_k2(
            x1, s1, q1,
            row(p['gin%d_bn_g' % l]), row(p['gin%d_bn_b' % l]),
            p['gin%d_w2' % l].T.astype(f32), row(p['gin%d_b2' % l]))
        h = _gin_k3(x2, s2, q2, row(p['bn%d_g' % l]), row(p['bn%d_b' % l]))

    gh = _att(h, p['att_k_w'].T.astype(f32), row(p['att_k_b']),
              p['att_q_w'].T.astype(f32), p['att_v_w'].T.astype(f32),
              row(p['att_v_b']))

    preds = _head(gh,
                  p['pc_w1'].T.astype(f32), row(p['pc_b1']),
                  row(p['pc_bn1_g']), row(p['pc_bn1_b']),
                  p['pc_w2'].T.astype(f32), row(p['pc_b2']),
                  row(p['pc_bn2_g']), row(p['pc_bn2_b']),
                  p['pc_w3'].T.astype(f32), row(p['pc_b3']))
    return preds
